# baseline (speedup 1.0000x reference)
import functools

import jax
import jax.numpy as jnp
from jax import lax
from jax.experimental import pallas as pl
from jax.experimental.pallas import tpu as pltpu
from jax.experimental.pallas import tpu_sc as plsc

NC = 2
NS = 16
NW = NC * NS
K1 = 64
K2 = 16

F32 = jnp.float32
I32 = jnp.int32


def _round_up(v, m):
    return ((v + m - 1) // m) * m


_GDN = lax.GatherDimensionNumbers(offset_dims=(), collapsed_slice_dims=(0,),
                                  start_index_map=(0,))


def _splat(vec, lane):
    idx = jnp.full((16, 1), lane, I32)
    return lax.gather(vec, idx, _GDN, (1,),
                      mode=lax.GatherScatterMode.PROMISE_IN_BOUNDS)


def _l1_prep_body(x_ref, w1_ref, atts_ref, attd_ref, sel_ref,
                  h1_ref, ts_ref, td_ref, m_ref):
    h = jnp.dot(x_ref[...], w1_ref[...], preferred_element_type=F32)
    h1_ref[...] = h
    a_s = jnp.dot(h * atts_ref[...], sel_ref[...], preferred_element_type=F32)
    a_d = jnp.dot(h * attd_ref[...], sel_ref[...], preferred_element_type=F32)
    ts_ref[...] = jnp.concatenate([a_s, a_s], axis=1)
    td_ref[...] = jnp.concatenate([a_d, a_d], axis=1)
    m = (jnp.max(a_s, axis=0, keepdims=True)
         + jnp.max(a_d, axis=0, keepdims=True))
    m_ref[...] = jnp.concatenate([m, m], axis=1)


def _l2_prep_body(o_ref, b1_ref, w2_ref, atts2_ref, attd2_ref, sel2_ref,
                  h2_ref, ts2_ref, td2_ref, ms_ref, md_ref):
    act = o_ref[0] + o_ref[1] + b1_ref[...]
    act = jnp.where(act > 0.0, act, jnp.exp(act) - 1.0)
    h2 = jnp.dot(act, w2_ref[...], preferred_element_type=F32)
    h2_ref[...] = h2
    a_s = jnp.dot(h2 * atts2_ref[...], sel2_ref[...],
                  preferred_element_type=F32)
    a_d = jnp.dot(h2 * attd2_ref[...], sel2_ref[...],
                  preferred_element_type=F32)
    ts2_ref[...] = jnp.concatenate([a_s, a_s], axis=1)
    td2_ref[...] = jnp.concatenate([a_d, a_d], axis=1)
    bs = jnp.max(a_s, axis=0, keepdims=True)
    bd = jnp.max(a_d, axis=0, keepdims=True)

    @pl.when(pl.program_id(0) == 0)
    def _():
        ms_ref[...] = bs
        md_ref[...] = bd

    @pl.when(pl.program_id(0) > 0)
    def _():
        ms_ref[...] = jnp.maximum(ms_ref[...], bs)
        md_ref[...] = jnp.maximum(md_ref[...], bd)


def _den_comb_body(d_ref, out_ref):
    out_ref[...] = 1.0 / (d_ref[0] + d_ref[1] + 1e-16)


def _final_body(q_ref, b2_ref, out_ref):
    out_ref[...] = (q_ref[0] + q_ref[1]) * 0.125 + b2_ref[...]


def _zero_rows(zbuf, shared, row0, nrows, zrows):
    full = nrows // zrows
    rem = nrows - full * zrows
    for r in range(full):
        pltpu.sync_copy(zbuf, shared.at[pl.ds(row0 + r * zrows, zrows), :])
    if rem:
        pltpu.sync_copy(zbuf.at[pl.ds(0, rem), :],
                        shared.at[pl.ds(row0 + full * zrows, rem), :])


def _copy_idx(dst, src, k):
    for j in range(k // 16):
        dst[pl.ds(16 * j, 16)] = src[pl.ds(16 * j, 16)]


def _make_scores_kernel(e_pad, np_):
    ept = e_pad // NW
    nch = ept // K1
    rpt = np_ // NS
    mesh = plsc.VectorSubcoreMesh(core_axis_name="c", subcore_axis_name="s",
                                  num_cores=NC, num_subcores=NS)

    @functools.partial(
        pl.kernel,
        out_type=(jax.ShapeDtypeStruct((e_pad, 16), F32),
                  jax.ShapeDtypeStruct((NC, np_, 16), F32)),
        mesh=mesh,
        compiler_params=pltpu.CompilerParams(use_tc_tiling_on_sc=False,
                                             needs_layout_passes=False),
        scratch_types=[
            [pltpu.VMEM((K1,), I32) for _ in range(2)],
            [pltpu.VMEM((K1,), I32) for _ in range(2)],
            pltpu.VMEM((2 * K1,), I32),
            [pltpu.VMEM((K1, 16), F32) for _ in range(2)],
            [pltpu.VMEM((K1, 16), F32) for _ in range(2)],
            pltpu.VMEM((2 * K1, 16), F32),
            pltpu.VMEM((16,), F32),
            pltpu.VMEM_SHARED((np_, 16), F32),
            [pltpu.SemaphoreType.DMA for _ in range(2)],
            [pltpu.SemaphoreType.DMA for _ in range(2)],
            [pltpu.SemaphoreType.DMA for _ in range(2)],
        ],
    )
    def scores_k(src_hbm, dst_hbm, ts_hbm, td_hbm, m_hbm,
                 ex_hbm, den_hbm,
                 srcv, dstv, scatv, sv, dv, exv, mv, den_sh,
                 sidx, sdat, ssc):
        cid = lax.axis_index("c")
        sid = lax.axis_index("s")
        t = cid * NS + sid

        def zloop(i, _):
            exv[i, :] = jnp.zeros((16,), F32)
            return 0
        lax.fori_loop(0, 2 * K1, zloop, 0)
        _zero_rows(exv, den_sh, sid * rpt, rpt, 2 * K1)
        plsc.subcore_barrier()
        pltpu.sync_copy(m_hbm.at[0], mv)
        m = mv[...]

        def base(c):
            return t * ept + c * K1

        def issue_idx(c, b):
            pltpu.async_copy(src_hbm.at[pl.ds(base(c), K1)], srcv[b], sidx[b])
            pltpu.async_copy(dst_hbm.at[pl.ds(base(c), K1)], dstv[b], sidx[b])

        def wait_idx(c, b):
            pltpu.make_async_copy(src_hbm.at[pl.ds(base(c), K1)], srcv[b],
                                  sidx[b]).wait()
            pltpu.make_async_copy(dst_hbm.at[pl.ds(base(c), K1)], dstv[b],
                                  sidx[b]).wait()

        def issue_data(c, b):
            pltpu.async_copy(ts_hbm.at[srcv[b]], sv[b], sdat[b])
            pltpu.async_copy(td_hbm.at[dstv[b]], dv[b], sdat[b])

        def wait_data(c, b):
            pltpu.make_async_copy(ts_hbm.at[srcv[b]], sv[b], sdat[b]).wait()
            pltpu.make_async_copy(td_hbm.at[dstv[b]], dv[b], sdat[b]).wait()

        def compute(c, b, j):
            def inner(i, _):
                e = sv[b][i, :] + dv[b][i, :]
                e = jnp.where(e > 0.0, e, 0.2 * e)
                exv[j * K1 + i, :] = jnp.exp(e - m)
                return 0
            lax.fori_loop(0, K1, inner, 0)

        def group_out(c0):
            pltpu.sync_copy(exv, ex_hbm.at[pl.ds(base(c0), 2 * K1), :])
            pltpu.sync_copy(exv, den_sh.at[scatv], add=True)

        def slot(c, b, j, do_next, do_idx2):
            if do_next:
                wait_idx(c + 1, b ^ 1)
                issue_data(c + 1, b ^ 1)
            wait_data(c, b)
            for jj in range(K1 // 16):
                scatv[pl.ds(j * K1 + 16 * jj, 16)] = (
                    dstv[b][pl.ds(16 * jj, 16)])
            if do_idx2:
                issue_idx(c + 2, b)
            compute(c, b, j)

        pltpu.sync_copy(src_hbm.at[pl.ds(base(0), K1)], srcv[0])
        pltpu.sync_copy(dst_hbm.at[pl.ds(base(0), K1)], dstv[0])
        issue_data(0, 0)
        issue_idx(1, 1)
        slot(0, 0, 0, True, True)
        slot(1, 1, 1, True, True)
        group_out(0)

        def steady(g, _):
            c = 2 * g
            slot(c, 0, 0, True, True)
            slot(c + 1, 1, 1, True, True)
            group_out(c)
            return 0
        lax.fori_loop(1, nch // 2 - 1, steady, 0)

        slot(nch - 2, 0, 0, True, False)
        slot(nch - 1, 1, 1, False, False)
        group_out(nch - 2)

        plsc.subcore_barrier()
        pltpu.sync_copy(den_sh.at[pl.ds(sid * rpt, rpt), :],
                        den_hbm.at[cid, pl.ds(sid * rpt, rpt), :])

    return scores_k


def _make_msg1_kernel(e_pad, np_):
    ept = e_pad // NW
    nch = ept // K1
    rpt = np_ // NS
    mesh = plsc.VectorSubcoreMesh(core_axis_name="c", subcore_axis_name="s",
                                  num_cores=NC, num_subcores=NS)

    @functools.partial(
        pl.kernel,
        out_type=jax.ShapeDtypeStruct((NC, np_, 128), F32),
        mesh=mesh,
        compiler_params=pltpu.CompilerParams(use_tc_tiling_on_sc=False,
                                             needs_layout_passes=False),
        scratch_types=[
            [pltpu.VMEM((K1,), I32) for _ in range(2)],
            [pltpu.VMEM((K1,), I32) for _ in range(2)],
            pltpu.VMEM((2 * K1,), I32),
            [pltpu.VMEM((K1, 16), F32) for _ in range(2)],
            [pltpu.VMEM((K1, 16), F32) for _ in range(2)],
            [pltpu.VMEM((K1, 128), F32) for _ in range(2)],
            pltpu.VMEM((2 * K1, 128), F32),
            pltpu.VMEM_SHARED((np_, 128), F32),
            [pltpu.SemaphoreType.DMA for _ in range(2)],
            [pltpu.SemaphoreType.DMA for _ in range(2)],
            [pltpu.SemaphoreType.DMA for _ in range(2)],
        ],
    )
    def msg1_k(src_hbm, dst_hbm, ex_hbm, den_hbm, h1_hbm,
               o_hbm,
               srcv, dstv, scatv, exv, denv, h1v, msgv, out_sh,
               sidx, sdat, ssc):
        cid = lax.axis_index("c")
        sid = lax.axis_index("s")
        t = cid * NS + sid

        def zloop(i, _):
            for k in range(8):
                msgv[i, pl.ds(16 * k, 16)] = jnp.zeros((16,), F32)
            return 0
        lax.fori_loop(0, 2 * K1, zloop, 0)
        _zero_rows(msgv, out_sh, sid * rpt, rpt, 2 * K1)
        plsc.subcore_barrier()

        def base(c):
            return t * ept + c * K1

        def issue_idx(c, b):
            pltpu.async_copy(src_hbm.at[pl.ds(base(c), K1)], srcv[b], sidx[b])
            pltpu.async_copy(dst_hbm.at[pl.ds(base(c), K1)], dstv[b], sidx[b])

        def wait_idx(c, b):
            pltpu.make_async_copy(src_hbm.at[pl.ds(base(c), K1)], srcv[b],
                                  sidx[b]).wait()
            pltpu.make_async_copy(dst_hbm.at[pl.ds(base(c), K1)], dstv[b],
                                  sidx[b]).wait()

        def issue_data(c, b):
            pltpu.async_copy(h1_hbm.at[srcv[b]], h1v[b], sdat[b])
            pltpu.async_copy(den_hbm.at[dstv[b]], denv[b], sdat[b])
            pltpu.async_copy(ex_hbm.at[pl.ds(base(c), K1), :], exv[b], sdat[b])

        def wait_data(c, b):
            pltpu.make_async_copy(h1_hbm.at[srcv[b]], h1v[b], sdat[b]).wait()
            pltpu.make_async_copy(den_hbm.at[dstv[b]], denv[b], sdat[b]).wait()
            pltpu.make_async_copy(ex_hbm.at[pl.ds(base(c), K1), :], exv[b],
                                  sdat[b]).wait()

        def compute(c, b, j):
            def inner(i, _):
                r = exv[b][i, :] * denv[b][i, :]
                for h in range(8):
                    a = _splat(r, h)
                    msgv[j * K1 + i, pl.ds(16 * h, 16)] = (
                        h1v[b][i, pl.ds(16 * h, 16)] * a)
                return 0
            lax.fori_loop(0, K1, inner, 0)

        def group_out():
            pltpu.sync_copy(msgv, out_sh.at[scatv], add=True)

        def slot(c, b, j, do_next, do_idx2):
            if do_next:
                wait_idx(c + 1, b ^ 1)
                issue_data(c + 1, b ^ 1)
            wait_data(c, b)
            for jj in range(K1 // 16):
                scatv[pl.ds(j * K1 + 16 * jj, 16)] = (
                    dstv[b][pl.ds(16 * jj, 16)])
            if do_idx2:
                issue_idx(c + 2, b)
            compute(c, b, j)

        pltpu.sync_copy(src_hbm.at[pl.ds(base(0), K1)], srcv[0])
        pltpu.sync_copy(dst_hbm.at[pl.ds(base(0), K1)], dstv[0])
        issue_data(0, 0)
        issue_idx(1, 1)
        slot(0, 0, 0, True, True)
        slot(1, 1, 1, True, True)
        group_out()

        def steady(g, _):
            c = 2 * g
            slot(c, 0, 0, True, True)
            slot(c + 1, 1, 1, True, True)
            group_out()
            return 0
        lax.fori_loop(1, nch // 2 - 1, steady, 0)

        slot(nch - 2, 0, 0, True, False)
        slot(nch - 1, 1, 1, False, False)
        group_out()

        plsc.subcore_barrier()
        pltpu.sync_copy(out_sh.at[pl.ds(sid * rpt, rpt), :],
                        o_hbm.at[cid, pl.ds(sid * rpt, rpt), :])

    return msg1_k


def _make_msg2_kernel(e_pad, np_):
    ept = e_pad // NW
    nch = ept // K2
    rpt = np_ // NS
    mesh = plsc.VectorSubcoreMesh(core_axis_name="c", subcore_axis_name="s",
                                  num_cores=NC, num_subcores=NS)

    @functools.partial(
        pl.kernel,
        out_type=jax.ShapeDtypeStruct((NC, np_, 128), F32),
        mesh=mesh,
        compiler_params=pltpu.CompilerParams(use_tc_tiling_on_sc=False,
                                             needs_layout_passes=False),
        scratch_types=[
            [pltpu.VMEM((K2,), I32) for _ in range(2)],
            [pltpu.VMEM((K2,), I32) for _ in range(2)],
            pltpu.VMEM((4 * K2,), I32),
            [pltpu.VMEM((K2, 16), F32) for _ in range(2)],
            [pltpu.VMEM((K2, 16), F32) for _ in range(2)],
            [pltpu.VMEM((K2, 1024), F32) for _ in range(2)],
            pltpu.VMEM((4 * K2, 128), F32),
            pltpu.VMEM_SHARED((np_, 128), F32),
            [pltpu.SemaphoreType.DMA for _ in range(2)],
            [pltpu.SemaphoreType.DMA for _ in range(2)],
            [pltpu.SemaphoreType.DMA for _ in range(2)],
        ],
    )
    def msg2_k(src_hbm, dst_hbm, ex_hbm, den_hbm, h2_hbm,
               q_hbm,
               srcv, dstv, scatv, exv, denv, bufv, msgv, out_sh,
               sidx, sdat, ssc):
        cid = lax.axis_index("c")
        sid = lax.axis_index("s")
        t = cid * NS + sid

        def zloop(i, _):
            for k in range(8):
                msgv[i, pl.ds(16 * k, 16)] = jnp.zeros((16,), F32)
            return 0
        lax.fori_loop(0, 4 * K2, zloop, 0)
        _zero_rows(msgv, out_sh, sid * rpt, rpt, 4 * K2)
        plsc.subcore_barrier()

        def base(c):
            return t * ept + c * K2

        def issue_idx(c, b):
            pltpu.async_copy(src_hbm.at[pl.ds(base(c), K2)], srcv[b], sidx[b])
            pltpu.async_copy(dst_hbm.at[pl.ds(base(c), K2)], dstv[b], sidx[b])

        def wait_idx(c, b):
            pltpu.make_async_copy(src_hbm.at[pl.ds(base(c), K2)], srcv[b],
                                  sidx[b]).wait()
            pltpu.make_async_copy(dst_hbm.at[pl.ds(base(c), K2)], dstv[b],
                                  sidx[b]).wait()

        def issue_data(c, b):
            pltpu.async_copy(h2_hbm.at[srcv[b]], bufv[b], sdat[b])
            pltpu.async_copy(den_hbm.at[dstv[b]], denv[b], sdat[b])
            pltpu.async_copy(ex_hbm.at[pl.ds(base(c), K2), :], exv[b], sdat[b])

        def wait_data(c, b):
            pltpu.make_async_copy(h2_hbm.at[srcv[b]], bufv[b], sdat[b]).wait()
            pltpu.make_async_copy(den_hbm.at[dstv[b]], denv[b], sdat[b]).wait()
            pltpu.make_async_copy(ex_hbm.at[pl.ds(base(c), K2), :], exv[b],
                                  sdat[b]).wait()

        def compute(c, b, j):
            def inner(i, _):
                r = exv[b][i, :] * denv[b][i, :]
                accs = [None] * 8
                for h in range(8):
                    a = _splat(r, h)
                    for k in range(8):
                        v = bufv[b][i, pl.ds(h * 128 + 16 * k, 16)] * a
                        accs[k] = v if h == 0 else accs[k] + v
                for k in range(8):
                    msgv[j * K2 + i, pl.ds(16 * k, 16)] = accs[k]
                return 0
            lax.fori_loop(0, K2, inner, 0)

        def group_out():
            pltpu.sync_copy(msgv, out_sh.at[scatv], add=True)

        def slot(c, b, j, do_next, do_idx2):
            if do_next:
                wait_idx(c + 1, b ^ 1)
                issue_data(c + 1, b ^ 1)
            wait_data(c, b)
            scatv[pl.ds(j * K2, 16)] = dstv[b][pl.ds(0, 16)]
            if do_idx2:
                issue_idx(c + 2, b)
            compute(c, b, j)

        pltpu.sync_copy(src_hbm.at[pl.ds(base(0), K2)], srcv[0])
        pltpu.sync_copy(dst_hbm.at[pl.ds(base(0), K2)], dstv[0])
        issue_data(0, 0)
        issue_idx(1, 1)
        for j in range(4):
            slot(j, j & 1, j, True, True)
        group_out()

        def steady(g, _):
            c = 4 * g
            for j in range(4):
                slot(c + j, j & 1, j, True, True)
            group_out()
            return 0
        lax.fori_loop(1, nch // 4 - 1, steady, 0)

        c0 = nch - 4
        slot(c0, 0, 0, True, True)
        slot(c0 + 1, 1, 1, True, True)
        slot(c0 + 2, 0, 2, True, False)
        slot(c0 + 3, 1, 3, False, False)
        group_out()

        plsc.subcore_barrier()
        pltpu.sync_copy(out_sh.at[pl.ds(sid * rpt, rpt), :],
                        q_hbm.at[cid, pl.ds(sid * rpt, rpt), :])

    return msg2_k


def kernel(x, edge_index, W1, att_src1, att_dst1, b1,
           W2, att_src2, att_dst2, b2):
    n, in_dim = x.shape
    e = edge_index.shape[1]
    np_ = _round_up(n + 1, 128)
    e_tot = e + n
    e_pad = _round_up(e_tot, NW * 128)

    loops = jnp.arange(n, dtype=I32)
    padv = jnp.full((e_pad - e_tot,), n, dtype=I32)
    src = jnp.concatenate([edge_index[0].astype(I32), loops, padv])
    dst = jnp.concatenate([edge_index[1].astype(I32), loops, padv])

    xp = jnp.zeros((np_, in_dim), F32).at[:n].set(x)
    atts1 = att_src1.reshape(1, -1)
    attd1 = att_dst1.reshape(1, -1)
    sel = (jnp.arange(128)[:, None] // 16 == jnp.arange(8)[None, :]
           ).astype(F32)
    atts2 = att_src2.reshape(1, -1)
    attd2 = att_dst2.reshape(1, -1)
    sel2 = (jnp.arange(1024)[:, None] // 128 == jnp.arange(8)[None, :]
            ).astype(F32)

    h1, ts1, td1, m1 = pl.pallas_call(
        _l1_prep_body,
        out_shape=[jax.ShapeDtypeStruct((np_, 128), F32),
                   jax.ShapeDtypeStruct((np_, 16), F32),
                   jax.ShapeDtypeStruct((np_, 16), F32),
                   jax.ShapeDtypeStruct((1, 16), F32)],
    )(xp, W1, atts1, attd1, sel)

    scores_k = _make_scores_kernel(e_pad, np_)
    msg1_k = _make_msg1_kernel(e_pad, np_)
    msg2_k = _make_msg2_kernel(e_pad, np_)

    ex1, den1p = scores_k(src, dst, ts1, td1, m1)
    den1 = pl.pallas_call(
        _den_comb_body,
        out_shape=jax.ShapeDtypeStruct((np_, 16), F32),
    )(den1p)
    o1 = msg1_k(src, dst, ex1, den1, h1)

    nb = np_ // 128
    res = pl.pallas_call(
        _l2_prep_body,
        grid=(nb,),
        in_specs=[pl.BlockSpec((2, 128, 128), lambda i: (0, i, 0)),
                  pl.BlockSpec((1, 128), lambda i: (0, 0)),
                  pl.BlockSpec((128, 1024), lambda i: (0, 0)),
                  pl.BlockSpec((1, 1024), lambda i: (0, 0)),
                  pl.BlockSpec((1, 1024), lambda i: (0, 0)),
                  pl.BlockSpec((1024, 8), lambda i: (0, 0))],
        out_specs=[pl.BlockSpec((128, 1024), lambda i: (i, 0)),
                   pl.BlockSpec((128, 16), lambda i: (i, 0)),
                   pl.BlockSpec((128, 16), lambda i: (i, 0)),
                   pl.BlockSpec((1, 8), lambda i: (0, 0)),
                   pl.BlockSpec((1, 8), lambda i: (0, 0))],
        out_shape=[jax.ShapeDtypeStruct((np_, 1024), F32),
                   jax.ShapeDtypeStruct((np_, 16), F32),
                   jax.ShapeDtypeStruct((np_, 16), F32),
                   jax.ShapeDtypeStruct((1, 8), F32),
                   jax.ShapeDtypeStruct((1, 8), F32)],
    )(o1, b1.reshape(1, -1), W2, atts2, attd2, sel2)
    h2, ts2, td2, ms2, md2 = res
    m2 = jnp.concatenate([ms2 + md2, ms2 + md2], axis=1)

    ex2, den2p = scores_k(src, dst, ts2, td2, m2)
    den2 = pl.pallas_call(
        _den_comb_body,
        out_shape=jax.ShapeDtypeStruct((np_, 16), F32),
    )(den2p)
    q = msg2_k(src, dst, ex2, den2, h2)

    out = pl.pallas_call(
        _final_body,
        out_shape=jax.ShapeDtypeStruct((np_, 128), F32),
    )(q, b2.reshape(1, -1))
    return out[:n]


# speedup vs baseline: 1.0043x; 1.0043x over previous
import functools

import jax
import jax.numpy as jnp
from jax import lax
from jax.experimental import pallas as pl
from jax.experimental.pallas import tpu as pltpu
from jax.experimental.pallas import tpu_sc as plsc

NC = 2
NS = 16
NW = NC * NS
K1 = 64
K2 = 16

F32 = jnp.float32
I32 = jnp.int32


def _round_up(v, m):
    return ((v + m - 1) // m) * m


_GDN = lax.GatherDimensionNumbers(offset_dims=(), collapsed_slice_dims=(0,),
                                  start_index_map=(0,))


def _splat(vec, lane):
    idx = jnp.full((16, 1), lane, I32)
    return lax.gather(vec, idx, _GDN, (1,),
                      mode=lax.GatherScatterMode.PROMISE_IN_BOUNDS)


def _l1_prep_body(x_ref, w1_ref, atts_ref, attd_ref, sel_ref,
                  h1_ref, ts_ref, td_ref, m_ref):
    h = jnp.dot(x_ref[...], w1_ref[...], preferred_element_type=F32)
    h1_ref[...] = h
    a_s = jnp.dot(h * atts_ref[...], sel_ref[...], preferred_element_type=F32)
    a_d = jnp.dot(h * attd_ref[...], sel_ref[...], preferred_element_type=F32)
    ts_ref[...] = jnp.concatenate([a_s, a_s], axis=1)
    td_ref[...] = jnp.concatenate([a_d, a_d], axis=1)
    m = (jnp.max(a_s, axis=0, keepdims=True)
         + jnp.max(a_d, axis=0, keepdims=True))
    m_ref[...] = jnp.concatenate([m, m], axis=1)


def _l2_prep_body(o_ref, b1_ref, w2_ref, atts2_ref, attd2_ref, sel2_ref,
                  h2_ref, ts2_ref, td2_ref, ms_ref, md_ref):
    act = o_ref[0] + o_ref[1] + b1_ref[...]
    act = jnp.where(act > 0.0, act, jnp.exp(act) - 1.0)
    h2 = jnp.dot(act, w2_ref[...], preferred_element_type=F32)
    h2_ref[...] = h2
    a_s = jnp.dot(h2 * atts2_ref[...], sel2_ref[...],
                  preferred_element_type=F32)
    a_d = jnp.dot(h2 * attd2_ref[...], sel2_ref[...],
                  preferred_element_type=F32)
    ts2_ref[...] = jnp.concatenate([a_s, a_s], axis=1)
    td2_ref[...] = jnp.concatenate([a_d, a_d], axis=1)
    bs = jnp.max(a_s, axis=0, keepdims=True)
    bd = jnp.max(a_d, axis=0, keepdims=True)

    @pl.when(pl.program_id(0) == 0)
    def _():
        ms_ref[...] = bs
        md_ref[...] = bd

    @pl.when(pl.program_id(0) > 0)
    def _():
        ms_ref[...] = jnp.maximum(ms_ref[...], bs)
        md_ref[...] = jnp.maximum(md_ref[...], bd)


def _den_comb_body(d_ref, out_ref):
    out_ref[...] = 1.0 / (d_ref[0] + d_ref[1] + 1e-16)


def _final_body(q_ref, b2_ref, out_ref):
    out_ref[...] = (q_ref[0] + q_ref[1]) * 0.125 + b2_ref[...]


def _zero_rows(zbuf, shared, row0, nrows, zrows):
    full = nrows // zrows
    rem = nrows - full * zrows
    for r in range(full):
        pltpu.sync_copy(zbuf, shared.at[pl.ds(row0 + r * zrows, zrows), :])
    if rem:
        pltpu.sync_copy(zbuf.at[pl.ds(0, rem), :],
                        shared.at[pl.ds(row0 + full * zrows, rem), :])


def _copy_idx(dst, src, k):
    for j in range(k // 16):
        dst[pl.ds(16 * j, 16)] = src[pl.ds(16 * j, 16)]


def _make_scores_kernel(e_pad, np_):
    ept = e_pad // NW
    nch = ept // K1
    rpt = np_ // NS
    mesh = plsc.VectorSubcoreMesh(core_axis_name="c", subcore_axis_name="s",
                                  num_cores=NC, num_subcores=NS)

    @functools.partial(
        pl.kernel,
        out_type=(jax.ShapeDtypeStruct((e_pad, 16), F32),
                  jax.ShapeDtypeStruct((NC, np_, 16), F32)),
        mesh=mesh,
        compiler_params=pltpu.CompilerParams(use_tc_tiling_on_sc=False,
                                             needs_layout_passes=False),
        scratch_types=[
            [pltpu.VMEM((K1,), I32) for _ in range(2)],
            [pltpu.VMEM((K1,), I32) for _ in range(2)],
            pltpu.VMEM((2 * K1,), I32),
            [pltpu.VMEM((K1, 16), F32) for _ in range(2)],
            [pltpu.VMEM((K1, 16), F32) for _ in range(2)],
            pltpu.VMEM((2 * K1, 16), F32),
            pltpu.VMEM((16,), F32),
            pltpu.VMEM_SHARED((np_, 16), F32),
            [pltpu.SemaphoreType.DMA for _ in range(2)],
            [pltpu.SemaphoreType.DMA for _ in range(2)],
            [pltpu.SemaphoreType.DMA for _ in range(2)],
        ],
    )
    def scores_k(src_hbm, dst_hbm, ts_hbm, td_hbm, m_hbm,
                 ex_hbm, den_hbm,
                 srcv, dstv, scatv, sv, dv, exv, mv, den_sh,
                 sidx, sdat, ssc):
        cid = lax.axis_index("c")
        sid = lax.axis_index("s")
        t = cid * NS + sid

        def zloop(i, _):
            exv[i, :] = jnp.zeros((16,), F32)
            return 0
        lax.fori_loop(0, 2 * K1, zloop, 0)
        _zero_rows(exv, den_sh, sid * rpt, rpt, 2 * K1)
        plsc.subcore_barrier()
        pltpu.sync_copy(m_hbm.at[0], mv)
        m = mv[...]

        def base(c):
            return t * ept + c * K1

        def issue_idx(c, b):
            pltpu.async_copy(src_hbm.at[pl.ds(base(c), K1)], srcv[b], sidx[b])
            pltpu.async_copy(dst_hbm.at[pl.ds(base(c), K1)], dstv[b], sidx[b])

        def wait_idx(c, b):
            pltpu.make_async_copy(src_hbm.at[pl.ds(base(c), K1)], srcv[b],
                                  sidx[b]).wait()
            pltpu.make_async_copy(dst_hbm.at[pl.ds(base(c), K1)], dstv[b],
                                  sidx[b]).wait()

        def issue_data(c, b):
            pltpu.async_copy(ts_hbm.at[srcv[b]], sv[b], sdat[b])
            pltpu.async_copy(td_hbm.at[dstv[b]], dv[b], sdat[b])

        def wait_data(c, b):
            pltpu.make_async_copy(ts_hbm.at[srcv[b]], sv[b], sdat[b]).wait()
            pltpu.make_async_copy(td_hbm.at[dstv[b]], dv[b], sdat[b]).wait()

        def compute(c, b, j):
            def inner(i, _):
                for u in range(4):
                    ii = 4 * i + u
                    e = sv[b][ii, :] + dv[b][ii, :]
                    e = jnp.where(e > 0.0, e, 0.2 * e)
                    exv[j * K1 + ii, :] = jnp.exp(e - m)
                return 0
            lax.fori_loop(0, K1 // 4, inner, 0)

        def group_out(c0):
            pltpu.sync_copy(exv, ex_hbm.at[pl.ds(base(c0), 2 * K1), :])
            pltpu.sync_copy(exv, den_sh.at[scatv], add=True)

        def slot(c, b, j, do_next, do_idx2):
            if do_next:
                wait_idx(c + 1, b ^ 1)
                issue_data(c + 1, b ^ 1)
            wait_data(c, b)
            for jj in range(K1 // 16):
                scatv[pl.ds(j * K1 + 16 * jj, 16)] = (
                    dstv[b][pl.ds(16 * jj, 16)])
            if do_idx2:
                issue_idx(c + 2, b)
            compute(c, b, j)

        pltpu.sync_copy(src_hbm.at[pl.ds(base(0), K1)], srcv[0])
        pltpu.sync_copy(dst_hbm.at[pl.ds(base(0), K1)], dstv[0])
        issue_data(0, 0)
        issue_idx(1, 1)
        slot(0, 0, 0, True, True)
        slot(1, 1, 1, True, True)
        group_out(0)

        def steady(g, _):
            c = 2 * g
            slot(c, 0, 0, True, True)
            slot(c + 1, 1, 1, True, True)
            group_out(c)
            return 0
        lax.fori_loop(1, nch // 2 - 1, steady, 0)

        slot(nch - 2, 0, 0, True, False)
        slot(nch - 1, 1, 1, False, False)
        group_out(nch - 2)

        plsc.subcore_barrier()
        pltpu.sync_copy(den_sh.at[pl.ds(sid * rpt, rpt), :],
                        den_hbm.at[cid, pl.ds(sid * rpt, rpt), :])

    return scores_k


def _make_msg1_kernel(e_pad, np_):
    ept = e_pad // NW
    nch = ept // K1
    rpt = np_ // NS
    mesh = plsc.VectorSubcoreMesh(core_axis_name="c", subcore_axis_name="s",
                                  num_cores=NC, num_subcores=NS)

    @functools.partial(
        pl.kernel,
        out_type=jax.ShapeDtypeStruct((NC, np_, 128), F32),
        mesh=mesh,
        compiler_params=pltpu.CompilerParams(use_tc_tiling_on_sc=False,
                                             needs_layout_passes=False),
        scratch_types=[
            [pltpu.VMEM((K1,), I32) for _ in range(2)],
            [pltpu.VMEM((K1,), I32) for _ in range(2)],
            pltpu.VMEM((2 * K1,), I32),
            [pltpu.VMEM((K1, 16), F32) for _ in range(2)],
            [pltpu.VMEM((K1, 16), F32) for _ in range(2)],
            [pltpu.VMEM((K1, 128), F32) for _ in range(2)],
            pltpu.VMEM((2 * K1, 128), F32),
            pltpu.VMEM_SHARED((np_, 128), F32),
            [pltpu.SemaphoreType.DMA for _ in range(2)],
            [pltpu.SemaphoreType.DMA for _ in range(2)],
            [pltpu.SemaphoreType.DMA for _ in range(2)],
        ],
    )
    def msg1_k(src_hbm, dst_hbm, ex_hbm, den_hbm, h1_hbm,
               o_hbm,
               srcv, dstv, scatv, exv, denv, h1v, msgv, out_sh,
               sidx, sdat, ssc):
        cid = lax.axis_index("c")
        sid = lax.axis_index("s")
        t = cid * NS + sid

        def zloop(i, _):
            for k in range(8):
                msgv[i, pl.ds(16 * k, 16)] = jnp.zeros((16,), F32)
            return 0
        lax.fori_loop(0, 2 * K1, zloop, 0)
        _zero_rows(msgv, out_sh, sid * rpt, rpt, 2 * K1)
        plsc.subcore_barrier()

        def base(c):
            return t * ept + c * K1

        def issue_idx(c, b):
            pltpu.async_copy(src_hbm.at[pl.ds(base(c), K1)], srcv[b], sidx[b])
            pltpu.async_copy(dst_hbm.at[pl.ds(base(c), K1)], dstv[b], sidx[b])

        def wait_idx(c, b):
            pltpu.make_async_copy(src_hbm.at[pl.ds(base(c), K1)], srcv[b],
                                  sidx[b]).wait()
            pltpu.make_async_copy(dst_hbm.at[pl.ds(base(c), K1)], dstv[b],
                                  sidx[b]).wait()

        def issue_data(c, b):
            pltpu.async_copy(h1_hbm.at[srcv[b]], h1v[b], sdat[b])
            pltpu.async_copy(den_hbm.at[dstv[b]], denv[b], sdat[b])
            pltpu.async_copy(ex_hbm.at[pl.ds(base(c), K1), :], exv[b], sdat[b])

        def wait_data(c, b):
            pltpu.make_async_copy(h1_hbm.at[srcv[b]], h1v[b], sdat[b]).wait()
            pltpu.make_async_copy(den_hbm.at[dstv[b]], denv[b], sdat[b]).wait()
            pltpu.make_async_copy(ex_hbm.at[pl.ds(base(c), K1), :], exv[b],
                                  sdat[b]).wait()

        def compute(c, b, j):
            def inner(i, _):
                for u in range(4):
                    ii = 4 * i + u
                    r = exv[b][ii, :] * denv[b][ii, :]
                    for h in range(8):
                        a = _splat(r, h)
                        msgv[j * K1 + ii, pl.ds(16 * h, 16)] = (
                            h1v[b][ii, pl.ds(16 * h, 16)] * a)
                return 0
            lax.fori_loop(0, K1 // 4, inner, 0)

        def group_out():
            pltpu.sync_copy(msgv, out_sh.at[scatv], add=True)

        def slot(c, b, j, do_next, do_idx2):
            if do_next:
                wait_idx(c + 1, b ^ 1)
                issue_data(c + 1, b ^ 1)
            wait_data(c, b)
            for jj in range(K1 // 16):
                scatv[pl.ds(j * K1 + 16 * jj, 16)] = (
                    dstv[b][pl.ds(16 * jj, 16)])
            if do_idx2:
                issue_idx(c + 2, b)
            compute(c, b, j)

        pltpu.sync_copy(src_hbm.at[pl.ds(base(0), K1)], srcv[0])
        pltpu.sync_copy(dst_hbm.at[pl.ds(base(0), K1)], dstv[0])
        issue_data(0, 0)
        issue_idx(1, 1)
        slot(0, 0, 0, True, True)
        slot(1, 1, 1, True, True)
        group_out()

        def steady(g, _):
            c = 2 * g
            slot(c, 0, 0, True, True)
            slot(c + 1, 1, 1, True, True)
            group_out()
            return 0
        lax.fori_loop(1, nch // 2 - 1, steady, 0)

        slot(nch - 2, 0, 0, True, False)
        slot(nch - 1, 1, 1, False, False)
        group_out()

        plsc.subcore_barrier()
        pltpu.sync_copy(out_sh.at[pl.ds(sid * rpt, rpt), :],
                        o_hbm.at[cid, pl.ds(sid * rpt, rpt), :])

    return msg1_k


def _make_msg2_kernel(e_pad, np_):
    ept = e_pad // NW
    nch = ept // K2
    rpt = np_ // NS
    mesh = plsc.VectorSubcoreMesh(core_axis_name="c", subcore_axis_name="s",
                                  num_cores=NC, num_subcores=NS)

    @functools.partial(
        pl.kernel,
        out_type=jax.ShapeDtypeStruct((NC, np_, 128), F32),
        mesh=mesh,
        compiler_params=pltpu.CompilerParams(use_tc_tiling_on_sc=False,
                                             needs_layout_passes=False),
        scratch_types=[
            [pltpu.VMEM((K2,), I32) for _ in range(2)],
            [pltpu.VMEM((K2,), I32) for _ in range(2)],
            pltpu.VMEM((4 * K2,), I32),
            [pltpu.VMEM((K2, 16), F32) for _ in range(2)],
            [pltpu.VMEM((K2, 16), F32) for _ in range(2)],
            [pltpu.VMEM((K2, 1024), F32) for _ in range(2)],
            pltpu.VMEM((4 * K2, 128), F32),
            pltpu.VMEM_SHARED((np_, 128), F32),
            [pltpu.SemaphoreType.DMA for _ in range(2)],
            [pltpu.SemaphoreType.DMA for _ in range(2)],
            [pltpu.SemaphoreType.DMA for _ in range(2)],
        ],
    )
    def msg2_k(src_hbm, dst_hbm, ex_hbm, den_hbm, h2_hbm,
               q_hbm,
               srcv, dstv, scatv, exv, denv, bufv, msgv, out_sh,
               sidx, sdat, ssc):
        cid = lax.axis_index("c")
        sid = lax.axis_index("s")
        t = cid * NS + sid

        def zloop(i, _):
            for k in range(8):
                msgv[i, pl.ds(16 * k, 16)] = jnp.zeros((16,), F32)
            return 0
        lax.fori_loop(0, 4 * K2, zloop, 0)
        _zero_rows(msgv, out_sh, sid * rpt, rpt, 4 * K2)
        plsc.subcore_barrier()

        def base(c):
            return t * ept + c * K2

        def issue_idx(c, b):
            pltpu.async_copy(src_hbm.at[pl.ds(base(c), K2)], srcv[b], sidx[b])
            pltpu.async_copy(dst_hbm.at[pl.ds(base(c), K2)], dstv[b], sidx[b])

        def wait_idx(c, b):
            pltpu.make_async_copy(src_hbm.at[pl.ds(base(c), K2)], srcv[b],
                                  sidx[b]).wait()
            pltpu.make_async_copy(dst_hbm.at[pl.ds(base(c), K2)], dstv[b],
                                  sidx[b]).wait()

        def issue_data(c, b):
            pltpu.async_copy(h2_hbm.at[srcv[b]], bufv[b], sdat[b])
            pltpu.async_copy(den_hbm.at[dstv[b]], denv[b], sdat[b])
            pltpu.async_copy(ex_hbm.at[pl.ds(base(c), K2), :], exv[b], sdat[b])

        def wait_data(c, b):
            pltpu.make_async_copy(h2_hbm.at[srcv[b]], bufv[b], sdat[b]).wait()
            pltpu.make_async_copy(den_hbm.at[dstv[b]], denv[b], sdat[b]).wait()
            pltpu.make_async_copy(ex_hbm.at[pl.ds(base(c), K2), :], exv[b],
                                  sdat[b]).wait()

        def compute(c, b, j):
            def inner(i, _):
                for u in range(2):
                    ii = 2 * i + u
                    r = exv[b][ii, :] * denv[b][ii, :]
                    accs = [None] * 8
                    for h in range(8):
                        a = _splat(r, h)
                        for k in range(8):
                            v = bufv[b][ii, pl.ds(h * 128 + 16 * k, 16)] * a
                            accs[k] = v if h == 0 else accs[k] + v
                    for k in range(8):
                        msgv[j * K2 + ii, pl.ds(16 * k, 16)] = accs[k]
                return 0
            lax.fori_loop(0, K2 // 2, inner, 0)

        def group_out():
            pltpu.sync_copy(msgv, out_sh.at[scatv], add=True)

        def slot(c, b, j, do_next, do_idx2):
            if do_next:
                wait_idx(c + 1, b ^ 1)
                issue_data(c + 1, b ^ 1)
            wait_data(c, b)
            scatv[pl.ds(j * K2, 16)] = dstv[b][pl.ds(0, 16)]
            if do_idx2:
                issue_idx(c + 2, b)
            compute(c, b, j)

        pltpu.sync_copy(src_hbm.at[pl.ds(base(0), K2)], srcv[0])
        pltpu.sync_copy(dst_hbm.at[pl.ds(base(0), K2)], dstv[0])
        issue_data(0, 0)
        issue_idx(1, 1)
        for j in range(4):
            slot(j, j & 1, j, True, True)
        group_out()

        def steady(g, _):
            c = 4 * g
            for j in range(4):
                slot(c + j, j & 1, j, True, True)
            group_out()
            return 0
        lax.fori_loop(1, nch // 4 - 1, steady, 0)

        c0 = nch - 4
        slot(c0, 0, 0, True, True)
        slot(c0 + 1, 1, 1, True, True)
        slot(c0 + 2, 0, 2, True, False)
        slot(c0 + 3, 1, 3, False, False)
        group_out()

        plsc.subcore_barrier()
        pltpu.sync_copy(out_sh.at[pl.ds(sid * rpt, rpt), :],
                        q_hbm.at[cid, pl.ds(sid * rpt, rpt), :])

    return msg2_k


def kernel(x, edge_index, W1, att_src1, att_dst1, b1,
           W2, att_src2, att_dst2, b2):
    n, in_dim = x.shape
    e = edge_index.shape[1]
    np_ = _round_up(n + 1, 128)
    e_tot = e + n
    e_pad = _round_up(e_tot, NW * 128)

    loops = jnp.arange(n, dtype=I32)
    padv = jnp.full((e_pad - e_tot,), n, dtype=I32)
    src = jnp.concatenate([edge_index[0].astype(I32), loops, padv])
    dst = jnp.concatenate([edge_index[1].astype(I32), loops, padv])

    xp = jnp.zeros((np_, in_dim), F32).at[:n].set(x)
    atts1 = att_src1.reshape(1, -1)
    attd1 = att_dst1.reshape(1, -1)
    sel = (jnp.arange(128)[:, None] // 16 == jnp.arange(8)[None, :]
           ).astype(F32)
    atts2 = att_src2.reshape(1, -1)
    attd2 = att_dst2.reshape(1, -1)
    sel2 = (jnp.arange(1024)[:, None] // 128 == jnp.arange(8)[None, :]
            ).astype(F32)

    h1, ts1, td1, m1 = pl.pallas_call(
        _l1_prep_body,
        out_shape=[jax.ShapeDtypeStruct((np_, 128), F32),
                   jax.ShapeDtypeStruct((np_, 16), F32),
                   jax.ShapeDtypeStruct((np_, 16), F32),
                   jax.ShapeDtypeStruct((1, 16), F32)],
    )(xp, W1, atts1, attd1, sel)

    scores_k = _make_scores_kernel(e_pad, np_)
    msg1_k = _make_msg1_kernel(e_pad, np_)
    msg2_k = _make_msg2_kernel(e_pad, np_)

    ex1, den1p = scores_k(src, dst, ts1, td1, m1)
    den1 = pl.pallas_call(
        _den_comb_body,
        out_shape=jax.ShapeDtypeStruct((np_, 16), F32),
    )(den1p)
    o1 = msg1_k(src, dst, ex1, den1, h1)

    nb = np_ // 128
    res = pl.pallas_call(
        _l2_prep_body,
        grid=(nb,),
        in_specs=[pl.BlockSpec((2, 128, 128), lambda i: (0, i, 0)),
                  pl.BlockSpec((1, 128), lambda i: (0, 0)),
                  pl.BlockSpec((128, 1024), lambda i: (0, 0)),
                  pl.BlockSpec((1, 1024), lambda i: (0, 0)),
                  pl.BlockSpec((1, 1024), lambda i: (0, 0)),
                  pl.BlockSpec((1024, 8), lambda i: (0, 0))],
        out_specs=[pl.BlockSpec((128, 1024), lambda i: (i, 0)),
                   pl.BlockSpec((128, 16), lambda i: (i, 0)),
                   pl.BlockSpec((128, 16), lambda i: (i, 0)),
                   pl.BlockSpec((1, 8), lambda i: (0, 0)),
                   pl.BlockSpec((1, 8), lambda i: (0, 0))],
        out_shape=[jax.ShapeDtypeStruct((np_, 1024), F32),
                   jax.ShapeDtypeStruct((np_, 16), F32),
                   jax.ShapeDtypeStruct((np_, 16), F32),
                   jax.ShapeDtypeStruct((1, 8), F32),
                   jax.ShapeDtypeStruct((1, 8), F32)],
    )(o1, b1.reshape(1, -1), W2, atts2, attd2, sel2)
    h2, ts2, td2, ms2, md2 = res
    m2 = jnp.concatenate([ms2 + md2, ms2 + md2], axis=1)

    ex2, den2p = scores_k(src, dst, ts2, td2, m2)
    den2 = pl.pallas_call(
        _den_comb_body,
        out_shape=jax.ShapeDtypeStruct((np_, 16), F32),
    )(den2p)
    q = msg2_k(src, dst, ex2, den2, h2)

    out = pl.pallas_call(
        _final_body,
        out_shape=jax.ShapeDtypeStruct((np_, 128), F32),
    )(q, b2.reshape(1, -1))
    return out[:n]
